# baseline (speedup 1.0000x reference)
import functools

import jax
import jax.numpy as jnp
from jax import lax
from jax.experimental import pallas as pl
from jax.experimental.pallas import tpu as pltpu
from jax.experimental.pallas import tpu_sc as plsc

_CHUNK = 128
_NBUF = 4


@functools.cache
def _build(total, hid, pos_rows):
    info = plsc.get_sparse_core_info()
    nc, ns = info.num_cores, info.num_subcores
    nw = nc * ns
    b_per_w = total // nw
    n_chunks = b_per_w // _CHUNK
    lanes = hid // 16
    mesh = plsc.VectorSubcoreMesh(core_axis_name="c", subcore_axis_name="s")

    @functools.partial(
        pl.kernel,
        out_type=jax.ShapeDtypeStruct((total, hid), jnp.float32),
        mesh=mesh,
        scratch_types=[
            pltpu.VMEM((b_per_w,), jnp.int32),
            pltpu.VMEM((2 * pos_rows, hid), jnp.float32),
            [pltpu.VMEM((_CHUNK, hid), jnp.float32) for _ in range(_NBUF)],
            [pltpu.SemaphoreType.DMA for _ in range(_NBUF)],
            [pltpu.SemaphoreType.DMA for _ in range(_NBUF)],
        ],
    )
    def emb_kernel(x_hbm, emb_hbm, pos_hbm, out_hbm, idx_v, pos_v, bufs, gsems, ssems):
        wid = lax.axis_index("s") * nc + lax.axis_index("c")
        base = wid * b_per_w
        pltpu.sync_copy(x_hbm.at[pl.ds(base, b_per_w)], idx_v)

        def issue_gather(j):
            b = j % _NBUF
            return pltpu.async_copy(
                emb_hbm.at[idx_v.at[pl.ds(j * _CHUNK, _CHUNK)]],
                bufs[b],
                gsems[b],
            )

        g = [None] * n_chunks
        s = [None] * n_chunks
        g[0] = issue_gather(0)
        g[1] = issue_gather(1)
        pltpu.sync_copy(pos_hbm, pos_v.at[pl.ds(0, pos_rows)])
        pltpu.sync_copy(pos_hbm, pos_v.at[pl.ds(pos_rows, pos_rows)])

        def add_pos(j):
            buf = bufs[j % _NBUF]
            p0 = (j * _CHUNK) % pos_rows

            def row(i, carry):
                for u in range(2):
                    r = i * 2 + u
                    for c in range(lanes):
                        sl = pl.ds(c * 16, 16)
                        plsc.addupdate(buf.at[r, sl], pos_v[p0 + r, sl])
                return carry

            lax.fori_loop(0, _CHUNK // 2, row, 0)

        for j in range(n_chunks):
            b = j % _NBUF
            if j >= _NBUF - 2:
                s[j - (_NBUF - 2)].wait()
            if j + 2 < n_chunks:
                g[j + 2] = issue_gather(j + 2)
            g[j].wait()
            add_pos(j)
            s[j] = pltpu.async_copy(
                bufs[b], out_hbm.at[pl.ds(base + j * _CHUNK, _CHUNK)], ssems[b]
            )
        for j in range(n_chunks - (_NBUF - 2), n_chunks):
            s[j].wait()

    return emb_kernel


def kernel(x, emb, pos_emb):
    batch, seq = x.shape
    x_flat = x.reshape(-1).astype(jnp.int32)
    out = _build(batch * seq, emb.shape[1], pos_emb.shape[0])(x_flat, emb, pos_emb)
    return out.reshape(batch, seq, emb.shape[1])


# speedup vs baseline: 1.0434x; 1.0434x over previous
import functools

import jax
import jax.numpy as jnp
from jax import lax
from jax.experimental import pallas as pl
from jax.experimental.pallas import tpu as pltpu
from jax.experimental.pallas import tpu_sc as plsc

_SPLIT = 104
_NBUF = 7
_PREF = 3


@functools.cache
def _build(total, hid, pos_rows):
    info = plsc.get_sparse_core_info()
    nc, ns = info.num_cores, info.num_subcores
    nw = nc * ns
    b_per_w = total // nw
    n_sub = 2 * (b_per_w // pos_rows)
    lanes = hid // 16
    mesh = plsc.VectorSubcoreMesh(core_axis_name="c", subcore_axis_name="s")

    def sub_off_len(k):
        off = (k // 2) * pos_rows + (k % 2) * _SPLIT
        ln = _SPLIT if k % 2 == 0 else pos_rows - _SPLIT
        return off, ln, (k % 2) * _SPLIT

    @functools.partial(
        pl.kernel,
        out_type=jax.ShapeDtypeStruct((total, hid), jnp.float32),
        mesh=mesh,
        scratch_types=[
            pltpu.VMEM((b_per_w,), jnp.int32),
            pltpu.VMEM((pos_rows, hid), jnp.float32),
            [pltpu.VMEM((_SPLIT, hid), jnp.float32) for _ in range(_NBUF)],
            [pltpu.SemaphoreType.DMA for _ in range(_NBUF)],
            [pltpu.SemaphoreType.DMA for _ in range(_NBUF)],
        ],
    )
    def emb_kernel(x_hbm, emb_hbm, pos_hbm, out_hbm, idx_v, pos_v, bufs, gsems, ssems):
        wid = lax.axis_index("s") * nc + lax.axis_index("c")
        base = wid * b_per_w
        pltpu.sync_copy(x_hbm.at[pl.ds(base, b_per_w)], idx_v)

        def issue_gather(k):
            off, ln, _ = sub_off_len(k)
            b = k % _NBUF
            return pltpu.async_copy(
                emb_hbm.at[idx_v.at[pl.ds(off, ln)]],
                bufs[b].at[pl.ds(0, ln)],
                gsems[b],
            )

        g = [None] * n_sub
        s = [None] * n_sub
        for k in range(_PREF):
            g[k] = issue_gather(k)
        pltpu.sync_copy(pos_hbm, pos_v)

        def add_pos(k):
            _, ln, po = sub_off_len(k)
            buf = bufs[k % _NBUF]

            def row(i, carry):
                for u in range(2):
                    r = i * 2 + u
                    for c in range(lanes):
                        sl = pl.ds(c * 16, 16)
                        plsc.addupdate(buf.at[r, sl], pos_v[po + r, sl])
                return carry

            lax.fori_loop(0, ln // 2, row, 0)

        def issue_store(k):
            off, ln, _ = sub_off_len(k)
            b = k % _NBUF
            return pltpu.async_copy(
                bufs[b].at[pl.ds(0, ln)],
                out_hbm.at[pl.ds(base + off, ln)],
                ssems[b],
            )

        for k in range(n_sub):
            if k >= _NBUF - _PREF:
                s[k - (_NBUF - _PREF)].wait()
            if k + _PREF < n_sub:
                g[k + _PREF] = issue_gather(k + _PREF)
            g[k].wait()
            add_pos(k)
            s[k] = issue_store(k)
        for k in range(n_sub - (_NBUF - _PREF), n_sub):
            s[k].wait()

    return emb_kernel


def kernel(x, emb, pos_emb):
    batch, seq = x.shape
    x_flat = x.reshape(-1).astype(jnp.int32)
    out = _build(batch * seq, emb.shape[1], pos_emb.shape[0])(x_flat, emb, pos_emb)
    return out.reshape(batch, seq, emb.shape[1])
